# baseline (speedup 1.0000x reference)
import functools

import jax
import jax.numpy as jnp
from jax import lax
from jax.experimental import pallas as pl
from jax.experimental.pallas import tpu as pltpu
from jax.experimental.pallas import tpu_sc as plsc

B = 16384
D = 128
NC = 2
NS = 16
NW = NC * NS
ROWS_PER_W = B // NW
CH = 128
NCHUNK = ROWS_PER_W // CH


def _sc_gather_both(u_emb, i_emb, uidx2d, iidx2d):
  mesh = plsc.VectorSubcoreMesh(core_axis_name="c", subcore_axis_name="s")

  @functools.partial(
      pl.kernel,
      mesh=mesh,
      out_type=[
          jax.ShapeDtypeStruct((B, D), jnp.float32),
          jax.ShapeDtypeStruct((B, D), jnp.float32),
      ],
      scratch_types=[
          pltpu.VMEM((NCHUNK, CH), jnp.int32),
          pltpu.VMEM((NCHUNK, CH), jnp.int32),
          pltpu.VMEM((CH, D), jnp.float32),
          pltpu.VMEM((CH, D), jnp.float32),
          pltpu.SemaphoreType.DMA,
          pltpu.SemaphoreType.DMA,
      ],
  )
  def k(uemb_hbm, iemb_hbm, uidx_hbm, iidx_hbm, ue_out, ie_out,
        uidx_v, iidx_v, urows_v, irows_v, usem, isem):
    wid = lax.axis_index("s") * NC + lax.axis_index("c")
    idx_row0 = wid * NCHUNK
    base = wid * ROWS_PER_W
    pltpu.sync_copy(uidx_hbm.at[pl.ds(idx_row0, NCHUNK)], uidx_v)
    pltpu.sync_copy(iidx_hbm.at[pl.ds(idx_row0, NCHUNK)], iidx_v)
    for j in range(NCHUNK):
      cu = pltpu.async_copy(uemb_hbm.at[uidx_v.at[j]], urows_v, usem)
      ci = pltpu.async_copy(iemb_hbm.at[iidx_v.at[j]], irows_v, isem)
      cu.wait()
      pltpu.sync_copy(urows_v, ue_out.at[pl.ds(base + j * CH, CH)])
      ci.wait()
      pltpu.sync_copy(irows_v, ie_out.at[pl.ds(base + j * CH, CH)])

  return k(u_emb, i_emb, uidx2d, iidx2d)


BLK = 2048


def _tc_head_body(ue_ref, ie_ref, w1a_ref, w1b_ref, b1_ref, w2_ref, b2_ref,
                  w3_ref, b3_ref, out_ref):
  ue = ue_ref[...]
  ie = ie_ref[...]
  h1 = jnp.maximum(
      ue @ w1a_ref[...] + ie @ w1b_ref[...] + b1_ref[...], 0.0)
  h2 = jnp.maximum(h1 @ w2_ref[...] + b2_ref[...], 0.0)
  mlp = jnp.sum(h2 * w3_ref[...], axis=1)
  mf = jnp.sum(ue * ie, axis=1)
  pred = mf + mlp + b3_ref[0]
  out_ref[...] = jax.nn.sigmoid(pred)


def _tc_head(ue, ie, W1, b1, W2, b2, W3, b3):
  w1a = W1[:D]
  w1b = W1[D:]
  b1r = b1.reshape(1, D)
  b2r = b2.reshape(1, 16)
  w3r = W3.reshape(1, 16)
  grid = B // BLK
  return pl.pallas_call(
      _tc_head_body,
      grid=(grid,),
      in_specs=[
          pl.BlockSpec((BLK, D), lambda i: (i, 0)),
          pl.BlockSpec((BLK, D), lambda i: (i, 0)),
          pl.BlockSpec((D, D), lambda i: (0, 0)),
          pl.BlockSpec((D, D), lambda i: (0, 0)),
          pl.BlockSpec((1, D), lambda i: (0, 0)),
          pl.BlockSpec((D, 16), lambda i: (0, 0)),
          pl.BlockSpec((1, 16), lambda i: (0, 0)),
          pl.BlockSpec((1, 16), lambda i: (0, 0)),
          pl.BlockSpec(memory_space=pltpu.SMEM),
      ],
      out_specs=pl.BlockSpec((BLK,), lambda i: (i,)),
      out_shape=jax.ShapeDtypeStruct((B,), jnp.float32),
      compiler_params=pltpu.CompilerParams(
          dimension_semantics=("arbitrary",),
      ),
  )(ue, ie, w1a, w1b, b1r, W2, b2r, w3r, b3)


def kernel(user, item, u_emb, i_emb, W1, b1, W2, b2, W3, b3):
  uidx2d = user[:, 0].reshape(B // CH, CH)
  iidx2d = item[:, 0].reshape(B // CH, CH)
  ue, ie = _sc_gather_both(u_emb, i_emb, uidx2d, iidx2d)
  return _tc_head(ue, ie, W1, b1, W2, b2, W3, b3)


# speedup vs baseline: 1.0386x; 1.0386x over previous
import functools

import jax
import jax.numpy as jnp
from jax import lax
from jax.experimental import pallas as pl
from jax.experimental.pallas import tpu as pltpu
from jax.experimental.pallas import tpu_sc as plsc

B = 16384
D = 128
NC = 2
NS = 16
NW = NC * NS
ROWS_PER_W = B // NW
CH = 128
NCHUNK = ROWS_PER_W // CH


def _sc_gather_both(u_emb, i_emb, uidx2d, iidx2d):
  mesh = plsc.VectorSubcoreMesh(core_axis_name="c", subcore_axis_name="s")

  @functools.partial(
      pl.kernel,
      mesh=mesh,
      out_type=[
          jax.ShapeDtypeStruct((B, D), jnp.float32),
          jax.ShapeDtypeStruct((B, D), jnp.float32),
      ],
      scratch_types=[
          pltpu.VMEM((NCHUNK, CH), jnp.int32),
          pltpu.VMEM((NCHUNK, CH), jnp.int32),
          pltpu.VMEM((CH, D), jnp.float32),
          pltpu.VMEM((CH, D), jnp.float32),
          pltpu.VMEM((CH, D), jnp.float32),
          pltpu.VMEM((CH, D), jnp.float32),
          pltpu.SemaphoreType.DMA,
          pltpu.SemaphoreType.DMA,
          pltpu.SemaphoreType.DMA,
          pltpu.SemaphoreType.DMA,
      ],
  )
  def k(uemb_hbm, iemb_hbm, uidx_hbm, iidx_hbm, ue_out, ie_out,
        uidx_v, iidx_v, ubuf0, ubuf1, ibuf0, ibuf1,
        usem0, usem1, isem0, isem1):
    wid = lax.axis_index("s") * NC + lax.axis_index("c")
    idx_row0 = wid * NCHUNK
    base = wid * ROWS_PER_W
    pltpu.sync_copy(uidx_hbm.at[pl.ds(idx_row0, NCHUNK)], uidx_v)
    pltpu.sync_copy(iidx_hbm.at[pl.ds(idx_row0, NCHUNK)], iidx_v)
    ubufs, usems = (ubuf0, ubuf1), (usem0, usem1)
    ibufs, isems = (ibuf0, ibuf1), (isem0, isem1)
    du = [pltpu.async_copy(uemb_hbm.at[uidx_v.at[j]], ubufs[j], usems[j])
          for j in range(2)]
    di = [pltpu.async_copy(iemb_hbm.at[iidx_v.at[j]], ibufs[j], isems[j])
          for j in range(2)]
    for j in range(NCHUNK):
      s = j % 2
      du[s].wait()
      pltpu.sync_copy(ubufs[s], ue_out.at[pl.ds(base + j * CH, CH)])
      if j + 2 < NCHUNK:
        du[s] = pltpu.async_copy(
            uemb_hbm.at[uidx_v.at[j + 2]], ubufs[s], usems[s])
      di[s].wait()
      pltpu.sync_copy(ibufs[s], ie_out.at[pl.ds(base + j * CH, CH)])
      if j + 2 < NCHUNK:
        di[s] = pltpu.async_copy(
            iemb_hbm.at[iidx_v.at[j + 2]], ibufs[s], isems[s])

  return k(u_emb, i_emb, uidx2d, iidx2d)


BLK = 2048


def _tc_head_body(ue_ref, ie_ref, w1a_ref, w1b_ref, b1_ref, w2_ref, b2_ref,
                  w3_ref, b3_ref, out_ref):
  ue = ue_ref[...]
  ie = ie_ref[...]
  h1 = jnp.maximum(
      ue @ w1a_ref[...] + ie @ w1b_ref[...] + b1_ref[...], 0.0)
  h2 = jnp.maximum(h1 @ w2_ref[...] + b2_ref[...], 0.0)
  mlp = jnp.sum(h2 * w3_ref[...], axis=1)
  mf = jnp.sum(ue * ie, axis=1)
  pred = mf + mlp + b3_ref[0]
  out_ref[...] = jax.nn.sigmoid(pred)


def _tc_head(ue, ie, W1, b1, W2, b2, W3, b3):
  w1a = W1[:D]
  w1b = W1[D:]
  b1r = b1.reshape(1, D)
  b2r = b2.reshape(1, 16)
  w3r = W3.reshape(1, 16)
  grid = B // BLK
  return pl.pallas_call(
      _tc_head_body,
      grid=(grid,),
      in_specs=[
          pl.BlockSpec((BLK, D), lambda i: (i, 0)),
          pl.BlockSpec((BLK, D), lambda i: (i, 0)),
          pl.BlockSpec((D, D), lambda i: (0, 0)),
          pl.BlockSpec((D, D), lambda i: (0, 0)),
          pl.BlockSpec((1, D), lambda i: (0, 0)),
          pl.BlockSpec((D, 16), lambda i: (0, 0)),
          pl.BlockSpec((1, 16), lambda i: (0, 0)),
          pl.BlockSpec((1, 16), lambda i: (0, 0)),
          pl.BlockSpec(memory_space=pltpu.SMEM),
      ],
      out_specs=pl.BlockSpec((BLK,), lambda i: (i,)),
      out_shape=jax.ShapeDtypeStruct((B,), jnp.float32),
      compiler_params=pltpu.CompilerParams(
          dimension_semantics=("arbitrary",),
      ),
  )(ue, ie, w1a, w1b, b1r, W2, b2r, w3r, b3)


def kernel(user, item, u_emb, i_emb, W1, b1, W2, b2, W3, b3):
  uidx2d = user[:, 0].reshape(B // CH, CH)
  iidx2d = item[:, 0].reshape(B // CH, CH)
  ue, ie = _sc_gather_both(u_emb, i_emb, uidx2d, iidx2d)
  return _tc_head(ue, ie, W1, b1, W2, b2, W3, b3)


# speedup vs baseline: 1.2562x; 1.2095x over previous
import functools

import jax
import jax.numpy as jnp
from jax import lax
from jax.experimental import pallas as pl
from jax.experimental.pallas import tpu as pltpu
from jax.experimental.pallas import tpu_sc as plsc

B = 16384
D = 128
NC = 2
NS = 16
NW = NC * NS
ROWS_PER_W = B // NW
CH = 128
NCHUNK = ROWS_PER_W // CH


def _sc_gather_both(u_emb, i_emb, uidx2d, iidx2d):
  mesh = plsc.VectorSubcoreMesh(core_axis_name="c", subcore_axis_name="s")

  @functools.partial(
      pl.kernel,
      mesh=mesh,
      out_type=[
          jax.ShapeDtypeStruct((B, D), jnp.float32),
          jax.ShapeDtypeStruct((B, D), jnp.float32),
      ],
      scratch_types=[
          pltpu.VMEM((NCHUNK, CH), jnp.int32),
          pltpu.VMEM((NCHUNK, CH), jnp.int32),
          pltpu.VMEM((CH, D), jnp.float32),
          pltpu.VMEM((CH, D), jnp.float32),
          pltpu.VMEM((CH, D), jnp.float32),
          pltpu.VMEM((CH, D), jnp.float32),
          pltpu.SemaphoreType.DMA,
          pltpu.SemaphoreType.DMA,
          pltpu.SemaphoreType.DMA,
          pltpu.SemaphoreType.DMA,
      ],
  )
  def k(uemb_hbm, iemb_hbm, uidx_hbm, iidx_hbm, ue_out, ie_out,
        uidx_v, iidx_v, ubuf0, ubuf1, ibuf0, ibuf1,
        usem0, usem1, isem0, isem1):
    wid = lax.axis_index("s") * NC + lax.axis_index("c")
    idx_row0 = wid * NCHUNK
    base = wid * ROWS_PER_W
    pltpu.sync_copy(uidx_hbm.at[pl.ds(idx_row0, NCHUNK)], uidx_v)
    pltpu.sync_copy(iidx_hbm.at[pl.ds(idx_row0, NCHUNK)], iidx_v)
    ubufs, usems = (ubuf0, ubuf1), (usem0, usem1)
    ibufs, isems = (ibuf0, ibuf1), (isem0, isem1)
    du = [pltpu.async_copy(uemb_hbm.at[uidx_v.at[j]], ubufs[j], usems[j])
          for j in range(2)]
    di = [pltpu.async_copy(iemb_hbm.at[iidx_v.at[j]], ibufs[j], isems[j])
          for j in range(2)]
    for j in range(NCHUNK):
      s = j % 2
      du[s].wait()
      pltpu.sync_copy(ubufs[s], ue_out.at[pl.ds(base + j * CH, CH)])
      if j + 2 < NCHUNK:
        du[s] = pltpu.async_copy(
            uemb_hbm.at[uidx_v.at[j + 2]], ubufs[s], usems[s])
      di[s].wait()
      pltpu.sync_copy(ibufs[s], ie_out.at[pl.ds(base + j * CH, CH)])
      if j + 2 < NCHUNK:
        di[s] = pltpu.async_copy(
            iemb_hbm.at[iidx_v.at[j + 2]], ibufs[s], isems[s])

  return k(u_emb, i_emb, uidx2d, iidx2d)


BLK = 2048


def _tc_head_body(ue_ref, ie_ref, w1a_ref, w1b_ref, b1_ref, w2_ref, b2_ref,
                  w3_ref, b3_ref, out_ref):
  ue = ue_ref[...]
  ie = ie_ref[...]
  h1 = jnp.maximum(
      ue @ w1a_ref[...] + ie @ w1b_ref[...] + b1_ref[...], 0.0)
  h2 = jnp.maximum(h1 @ w2_ref[...] + b2_ref[...], 0.0)
  mlp = h2 @ w3_ref[...]
  mf = jnp.sum(ue * ie, axis=1, keepdims=True)
  pred_col = mf + mlp + b3_ref[0]
  pred_row = jnp.transpose(pred_col)
  out_ref[...] = jax.nn.sigmoid(pred_row)[None]


def _tc_head(ue, ie, W1, b1, W2, b2, W3, b3):
  w1a = W1[:D]
  w1b = W1[D:]
  b1r = b1.reshape(1, D)
  b2r = b2.reshape(1, 16)
  grid = B // BLK
  out2d = pl.pallas_call(
      _tc_head_body,
      grid=(grid,),
      in_specs=[
          pl.BlockSpec((BLK, D), lambda i: (i, 0)),
          pl.BlockSpec((BLK, D), lambda i: (i, 0)),
          pl.BlockSpec((D, D), lambda i: (0, 0)),
          pl.BlockSpec((D, D), lambda i: (0, 0)),
          pl.BlockSpec((1, D), lambda i: (0, 0)),
          pl.BlockSpec((D, 16), lambda i: (0, 0)),
          pl.BlockSpec((1, 16), lambda i: (0, 0)),
          pl.BlockSpec((16, 1), lambda i: (0, 0)),
          pl.BlockSpec(memory_space=pltpu.SMEM),
      ],
      out_specs=pl.BlockSpec((1, 1, BLK), lambda i: (i, 0, 0)),
      out_shape=jax.ShapeDtypeStruct((grid, 1, BLK), jnp.float32),
      compiler_params=pltpu.CompilerParams(
          dimension_semantics=("arbitrary",),
      ),
  )(ue, ie, w1a, w1b, b1r, W2, b2r, W3, b3)
  return out2d.reshape(B)


def kernel(user, item, u_emb, i_emb, W1, b1, W2, b2, W3, b3):
  uidx2d = user[:, 0].reshape(B // CH, CH)
  iidx2d = item[:, 0].reshape(B // CH, CH)
  ue, ie = _sc_gather_both(u_emb, i_emb, uidx2d, iidx2d)
  return _tc_head(ue, ie, W1, b1, W2, b2, W3, b3)


# speedup vs baseline: 1.3001x; 1.0350x over previous
import functools

import jax
import jax.numpy as jnp
from jax import lax
from jax.experimental import pallas as pl
from jax.experimental.pallas import tpu as pltpu
from jax.experimental.pallas import tpu_sc as plsc

B = 16384
D = 128
NC = 2
NS = 16
NW = NC * NS
ROWS_PER_W = B // NW
CH = 128
NCHUNK = ROWS_PER_W // CH


def _sc_gather_both(u_emb, i_emb, uidx2d, iidx2d):
  mesh = plsc.VectorSubcoreMesh(core_axis_name="c", subcore_axis_name="s")

  @functools.partial(
      pl.kernel,
      mesh=mesh,
      out_type=[
          jax.ShapeDtypeStruct((B, D), jnp.float32),
          jax.ShapeDtypeStruct((B, D), jnp.float32),
      ],
      scratch_types=[
          pltpu.VMEM((NCHUNK, CH), jnp.int32),
          pltpu.VMEM((NCHUNK, CH), jnp.int32),
          pltpu.VMEM((CH, D), jnp.float32),
          pltpu.VMEM((CH, D), jnp.float32),
          pltpu.VMEM((CH, D), jnp.float32),
          pltpu.VMEM((CH, D), jnp.float32),
          pltpu.SemaphoreType.DMA,
          pltpu.SemaphoreType.DMA,
          pltpu.SemaphoreType.DMA,
          pltpu.SemaphoreType.DMA,
      ],
  )
  def k(uemb_hbm, iemb_hbm, uidx_hbm, iidx_hbm, ue_out, ie_out,
        uidx_v, iidx_v, ubuf0, ubuf1, ibuf0, ibuf1,
        usem0, usem1, isem0, isem1):
    wid = lax.axis_index("s") * NC + lax.axis_index("c")
    idx_row0 = wid * NCHUNK
    base = wid * ROWS_PER_W
    pltpu.sync_copy(uidx_hbm.at[pl.ds(idx_row0, NCHUNK)], uidx_v)
    pltpu.sync_copy(iidx_hbm.at[pl.ds(idx_row0, NCHUNK)], iidx_v)
    ubufs, usems = (ubuf0, ubuf1), (usem0, usem1)
    ibufs, isems = (ibuf0, ibuf1), (isem0, isem1)
    du = [pltpu.async_copy(uemb_hbm.at[uidx_v.at[j]], ubufs[j], usems[j])
          for j in range(2)]
    di = [pltpu.async_copy(iemb_hbm.at[iidx_v.at[j]], ibufs[j], isems[j])
          for j in range(2)]
    for j in range(NCHUNK):
      s = j % 2
      du[s].wait()
      pltpu.sync_copy(ubufs[s], ue_out.at[pl.ds(base + j * CH, CH)])
      if j + 2 < NCHUNK:
        du[s] = pltpu.async_copy(
            uemb_hbm.at[uidx_v.at[j + 2]], ubufs[s], usems[s])
      di[s].wait()
      pltpu.sync_copy(ibufs[s], ie_out.at[pl.ds(base + j * CH, CH)])
      if j + 2 < NCHUNK:
        di[s] = pltpu.async_copy(
            iemb_hbm.at[iidx_v.at[j + 2]], ibufs[s], isems[s])

  return k(u_emb, i_emb, uidx2d, iidx2d)


BLK = 8192


def _tc_head_body(ue_ref, ie_ref, w1a_ref, w1b_ref, b1_ref, w2_ref, b2_ref,
                  w3_ref, b3_ref, out_ref):
  ue = ue_ref[...]
  ie = ie_ref[...]
  h1 = jnp.maximum(
      ue @ w1a_ref[...] + ie @ w1b_ref[...] + b1_ref[...], 0.0)
  h2 = jnp.maximum(h1 @ w2_ref[...] + b2_ref[...], 0.0)
  mlp = h2 @ w3_ref[...]
  mf = jnp.sum(ue * ie, axis=1, keepdims=True)
  pred_col = mf + mlp + b3_ref[0]
  pred_row = jnp.transpose(pred_col)
  out_ref[...] = jax.nn.sigmoid(pred_row)[None]


def _tc_head(ue, ie, W1, b1, W2, b2, W3, b3):
  w1a = W1[:D]
  w1b = W1[D:]
  b1r = b1.reshape(1, D)
  b2r = b2.reshape(1, 16)
  grid = B // BLK
  out2d = pl.pallas_call(
      _tc_head_body,
      grid=(grid,),
      in_specs=[
          pl.BlockSpec((BLK, D), lambda i: (i, 0)),
          pl.BlockSpec((BLK, D), lambda i: (i, 0)),
          pl.BlockSpec((D, D), lambda i: (0, 0)),
          pl.BlockSpec((D, D), lambda i: (0, 0)),
          pl.BlockSpec((1, D), lambda i: (0, 0)),
          pl.BlockSpec((D, 16), lambda i: (0, 0)),
          pl.BlockSpec((1, 16), lambda i: (0, 0)),
          pl.BlockSpec((16, 1), lambda i: (0, 0)),
          pl.BlockSpec(memory_space=pltpu.SMEM),
      ],
      out_specs=pl.BlockSpec((1, 1, BLK), lambda i: (i, 0, 0)),
      out_shape=jax.ShapeDtypeStruct((grid, 1, BLK), jnp.float32),
      compiler_params=pltpu.CompilerParams(
          dimension_semantics=("arbitrary",),
      ),
  )(ue, ie, w1a, w1b, b1r, W2, b2r, W3, b3)
  return out2d.reshape(B)


def kernel(user, item, u_emb, i_emb, W1, b1, W2, b2, W3, b3):
  uidx2d = user[:, 0].reshape(B // CH, CH)
  iidx2d = item[:, 0].reshape(B // CH, CH)
  ue, ie = _sc_gather_both(u_emb, i_emb, uidx2d, iidx2d)
  return _tc_head(ue, ie, W1, b1, W2, b2, W3, b3)
